# baseline (speedup 1.0000x reference)
import functools

import jax
import jax.numpy as jnp
from jax import lax
from jax.experimental import pallas as pl
from jax.experimental.pallas import tpu as pltpu
from jax.experimental.pallas import tpu_sc as plsc

N_NODES = 10000
N_PAD = 10240
N_EDGES = 320000
D = 128
CW = 16

NC = 2
NS = 16
NW = NC * NS
EDGES_PER_TILE = N_EDGES // NW
CHUNK = 50
CHUNKS_PER_TILE = EDGES_PER_TILE // CHUNK
ROWS_PER_TILE = N_PAD // NS
NBUF = 4
BM = 400

_MESH = plsc.VectorSubcoreMesh(core_axis_name="c", subcore_axis_name="s")
_SC_PARAMS = pltpu.CompilerParams(use_tc_tiling_on_sc=False)


def _tile_ids():
    c = lax.axis_index("c")
    s = lax.axis_index("s")
    return c, s, s * NC + c


def _stage_idx(idx_hbm, idx_v, wid):
    pltpu.sync_copy(idx_hbm.at[pl.ds(wid * CHUNKS_PER_TILE, CHUNKS_PER_TILE)],
                    idx_v)


def _make_sc_segsum(with_cnt):
    out_types = [jax.ShapeDtypeStruct((NC, N_PAD, D), jnp.bfloat16)]
    scratch = [
        pltpu.VMEM((CHUNKS_PER_TILE, CHUNK), jnp.int32),
        pltpu.VMEM((CHUNKS_PER_TILE, CHUNK), jnp.int32),
        [pltpu.VMEM((CHUNK, D), jnp.bfloat16)] * NBUF,
        pltpu.VMEM_SHARED((N_PAD, D), jnp.bfloat16),
        [pltpu.SemaphoreType.DMA] * NBUF,
    ]
    if with_cnt:
        out_types.append(jax.ShapeDtypeStruct((NC, N_PAD, CW), jnp.float32))
        scratch += [
            pltpu.VMEM((CHUNK, CW), jnp.float32),
            pltpu.VMEM_SHARED((N_PAD, CW), jnp.float32),
        ]

    @functools.partial(
        pl.kernel,
        mesh=_MESH,
        compiler_params=_SC_PARAMS,
        out_type=out_types,
        scratch_types=scratch,
    )
    def sc_segsum(table_hbm, src_hbm, dst_hbm, zfeat_hbm, *refs):
        if with_cnt:
            (zcnt_hbm, ones_hbm, out_hbm, cnt_hbm,
             src_v, dst_v, rows, acc, sems, ones_v, cacc) = refs
        else:
            out_hbm, src_v, dst_v, rows, acc, sems = refs
        c, s, wid = _tile_ids()
        row0 = s * ROWS_PER_TILE
        _stage_idx(src_hbm, src_v, wid)
        _stage_idx(dst_hbm, dst_v, wid)
        pltpu.sync_copy(zfeat_hbm.at[pl.ds(row0, ROWS_PER_TILE)],
                        acc.at[pl.ds(row0, ROWS_PER_TILE)])
        if with_cnt:
            pltpu.sync_copy(ones_hbm, ones_v)
            pltpu.sync_copy(zcnt_hbm.at[pl.ds(row0, ROWS_PER_TILE)],
                            cacc.at[pl.ds(row0, ROWS_PER_TILE)])
        plsc.subcore_barrier()

        for b in range(NBUF - 1):
            pltpu.async_copy(table_hbm.at[src_v.at[b]], rows[b], sems[b])

        def _group(i, carry):
            for b in range(NBUF):
                j = i * NBUF + b
                jn = j + NBUF - 1
                bn = (NBUF - 1 + b) % NBUF

                @pl.when(jn < CHUNKS_PER_TILE)
                def _():
                    pltpu.async_copy(table_hbm.at[src_v.at[jn]], rows[bn],
                                     sems[bn])
                pltpu.make_async_copy(table_hbm.at[src_v.at[j]], rows[b],
                                      sems[b]).wait()
                pltpu.sync_copy(rows[b], acc.at[dst_v.at[j]], add=True)
                if with_cnt:
                    pltpu.sync_copy(ones_v, cacc.at[dst_v.at[j]], add=True)
            return carry
        lax.fori_loop(0, CHUNKS_PER_TILE // NBUF, _group, 0)
        plsc.subcore_barrier()

        pltpu.sync_copy(acc.at[pl.ds(row0, ROWS_PER_TILE)],
                        out_hbm.at[c, pl.ds(row0, ROWS_PER_TILE)])
        if with_cnt:
            pltpu.sync_copy(cacc.at[pl.ds(row0, ROWS_PER_TILE)],
                            cnt_hbm.at[c, pl.ds(row0, ROWS_PER_TILE)])

    return sc_segsum


_sc_segsum_cnt = _make_sc_segsum(True)
_sc_segsum = _make_sc_segsum(False)


def _tc1_body(x_ref, s_ref, c_ref, At_ref, Bt_ref, b_ref, o_ref):
    cnt = c_ref[0][:, 0:1] + c_ref[1][:, 0:1]
    sblk = (s_ref[0][...].astype(jnp.float32)
            + s_ref[1][...].astype(jnp.float32))
    inv = jnp.where(cnt > 0.0, 1.0 / jnp.maximum(cnt, 1.0), 0.0)
    mm = jnp.dot(x_ref[...], At_ref[...], preferred_element_type=jnp.float32)
    mm2 = jnp.dot(sblk * inv, Bt_ref[...], preferred_element_type=jnp.float32)
    o_ref[...] = jnp.where(cnt > 0.0, mm + b_ref[...] + mm2,
                           0.0).astype(jnp.bfloat16)


def _tc2_body(h_ref, s_ref, c_ref, x_ref, At_ref, Bt_ref, b_ref, o_ref):
    cnt = c_ref[0][:, 0:1] + c_ref[1][:, 0:1]
    sblk = (s_ref[0][...].astype(jnp.float32)
            + s_ref[1][...].astype(jnp.float32))
    inv = jnp.where(cnt > 0.0, 1.0 / jnp.maximum(cnt, 1.0), 0.0)
    h32 = h_ref[...].astype(jnp.float32)
    mm = jnp.dot(h32, At_ref[...], preferred_element_type=jnp.float32)
    mm2 = jnp.dot(sblk * inv, Bt_ref[...], preferred_element_type=jnp.float32)
    o_ref[...] = jnp.where(cnt > 0.0, mm + b_ref[...] + mm2, 0.0) + x_ref[...]


_W_SPEC = pl.BlockSpec((D, D), lambda i: (0, 0))
_B_SPEC = pl.BlockSpec((1, D), lambda i: (0, 0))
_ROW_SPEC = pl.BlockSpec((BM, D), lambda i: (i, 0))
_S_SPEC = pl.BlockSpec((NC, BM, D), lambda i: (0, i, 0))
_C_SPEC = pl.BlockSpec((NC, BM, CW), lambda i: (0, i, 0))


def _tc1(x, s1, cnt_p, At, Bt, b2d):
    return pl.pallas_call(
        _tc1_body,
        grid=(N_NODES // BM,),
        in_specs=[_ROW_SPEC, _S_SPEC, _C_SPEC, _W_SPEC, _W_SPEC, _B_SPEC],
        out_specs=_ROW_SPEC,
        out_shape=jax.ShapeDtypeStruct((N_NODES, D), jnp.bfloat16),
    )(x, s1, cnt_p, At, Bt, b2d)


def _tc2(h, s2, cnt_p, x, At, Bt, b2d):
    return pl.pallas_call(
        _tc2_body,
        grid=(N_NODES // BM,),
        in_specs=[_ROW_SPEC, _S_SPEC, _C_SPEC, _ROW_SPEC,
                  _W_SPEC, _W_SPEC, _B_SPEC],
        out_specs=_ROW_SPEC,
        out_shape=jax.ShapeDtypeStruct((N_NODES, D), jnp.float32),
    )(h, s2, cnt_p, x, At, Bt, b2d)


def kernel(nodes_feat, edges_index, batch, W_in, b_in, W_out, b_out):
    x = nodes_feat
    src2d = edges_index[0].reshape(N_EDGES // CHUNK, CHUNK)
    dst2d = edges_index[1].reshape(N_EDGES // CHUNK, CHUNK)
    zfeat = jnp.zeros((N_PAD, D), jnp.bfloat16)
    zcnt = jnp.zeros((N_PAD, CW), jnp.float32)
    ones2d = jnp.ones((CHUNK, CW), jnp.float32)
    A1t = (W_in[:, :D] - W_in[:, D:]).T
    B1t = W_in[:, D:].T
    A2t = (W_out[:, :D] - W_out[:, D:]).T
    B2t = W_out[:, D:].T

    s1, cnt_p = _sc_segsum_cnt(x.astype(jnp.bfloat16), src2d, dst2d,
                               zfeat, zcnt, ones2d)
    h = _tc1(x, s1, cnt_p, A1t, B1t, b_in.reshape(1, D))
    (s2,) = _sc_segsum(h, src2d, dst2d, zfeat)
    return _tc2(h, s2, cnt_p, x, A2t, B2t, b_out.reshape(1, D))


# speedup vs baseline: 1.1477x; 1.1477x over previous
import functools

import jax
import jax.numpy as jnp
from jax import lax
from jax.experimental import pallas as pl
from jax.experimental.pallas import tpu as pltpu
from jax.experimental.pallas import tpu_sc as plsc

N_NODES = 10000
N_PAD = 10240
N_EDGES = 320000
D = 128
CW = 16

NC = 2
NS = 16
NW = NC * NS
EDGES_PER_TILE = N_EDGES // NW
CHUNK = 125
CHUNKS_PER_TILE = EDGES_PER_TILE // CHUNK
ROWS_PER_TILE = N_PAD // NS
NBUF = 4
BM = 400

_MESH = plsc.VectorSubcoreMesh(core_axis_name="c", subcore_axis_name="s")
_SC_PARAMS = pltpu.CompilerParams(use_tc_tiling_on_sc=False)


def _tile_ids():
    c = lax.axis_index("c")
    s = lax.axis_index("s")
    return c, s, s * NC + c


def _stage_idx(idx_hbm, idx_v, wid):
    pltpu.sync_copy(idx_hbm.at[pl.ds(wid * CHUNKS_PER_TILE, CHUNKS_PER_TILE)],
                    idx_v)


def _make_sc_segsum(with_cnt):
    out_types = [jax.ShapeDtypeStruct((NC, N_PAD, D), jnp.bfloat16)]
    scratch = [
        pltpu.VMEM((CHUNKS_PER_TILE, CHUNK), jnp.int32),
        pltpu.VMEM((CHUNKS_PER_TILE, CHUNK), jnp.int32),
        [pltpu.VMEM((CHUNK, D), jnp.bfloat16)] * NBUF,
        pltpu.VMEM_SHARED((N_PAD, D), jnp.bfloat16),
        [pltpu.SemaphoreType.DMA] * NBUF,
        [pltpu.SemaphoreType.DMA] * NBUF,
    ]
    if with_cnt:
        out_types.append(jax.ShapeDtypeStruct((NC, N_PAD, CW), jnp.float32))
        scratch += [
            pltpu.VMEM((CHUNK, CW), jnp.float32),
            pltpu.VMEM_SHARED((N_PAD, CW), jnp.float32),
        ]

    @functools.partial(
        pl.kernel,
        mesh=_MESH,
        compiler_params=_SC_PARAMS,
        out_type=out_types,
        scratch_types=scratch,
    )
    def sc_segsum(table_hbm, src_hbm, dst_hbm, zfeat_hbm, *refs):
        if with_cnt:
            (zcnt_hbm, ones_hbm, out_hbm, cnt_hbm,
             src_v, dst_v, rows, acc, gs, ss, ones_v, cacc) = refs
        else:
            out_hbm, src_v, dst_v, rows, acc, gs, ss = refs
        c, s, wid = _tile_ids()
        row0 = s * ROWS_PER_TILE
        _stage_idx(src_hbm, src_v, wid)
        _stage_idx(dst_hbm, dst_v, wid)
        pltpu.sync_copy(zfeat_hbm.at[pl.ds(row0, ROWS_PER_TILE)],
                        acc.at[pl.ds(row0, ROWS_PER_TILE)])
        if with_cnt:
            pltpu.sync_copy(ones_hbm, ones_v)
            pltpu.sync_copy(zcnt_hbm.at[pl.ds(row0, ROWS_PER_TILE)],
                            cacc.at[pl.ds(row0, ROWS_PER_TILE)])
        plsc.subcore_barrier()

        for b in range(NBUF - 1):
            pltpu.async_copy(table_hbm.at[src_v.at[b]], rows[b], gs[b])

        def _group(i, carry):
            for b in range(NBUF):
                j = i * NBUF + b
                jn = j + NBUF - 1
                bn = (NBUF - 1 + b) % NBUF

                @pl.when(j > 0)
                def _():
                    pltpu.make_async_copy(rows[bn], acc.at[dst_v.at[j - 1]],
                                          ss[bn]).wait()

                @pl.when(jn < CHUNKS_PER_TILE)
                def _():
                    pltpu.async_copy(table_hbm.at[src_v.at[jn]], rows[bn],
                                     gs[bn])
                pltpu.make_async_copy(table_hbm.at[src_v.at[j]], rows[b],
                                      gs[b]).wait()
                pltpu.async_copy(rows[b], acc.at[dst_v.at[j]], ss[b], add=True)
                if with_cnt:
                    pltpu.sync_copy(ones_v, cacc.at[dst_v.at[j]], add=True)
            return carry
        lax.fori_loop(0, CHUNKS_PER_TILE // NBUF, _group, 0)
        lastb = (CHUNKS_PER_TILE - 1) % NBUF
        pltpu.make_async_copy(rows[lastb],
                              acc.at[dst_v.at[CHUNKS_PER_TILE - 1]],
                              ss[lastb]).wait()
        plsc.subcore_barrier()

        pltpu.sync_copy(acc.at[pl.ds(row0, ROWS_PER_TILE)],
                        out_hbm.at[c, pl.ds(row0, ROWS_PER_TILE)])
        if with_cnt:
            pltpu.sync_copy(cacc.at[pl.ds(row0, ROWS_PER_TILE)],
                            cnt_hbm.at[c, pl.ds(row0, ROWS_PER_TILE)])

    return sc_segsum


_sc_segsum_cnt = _make_sc_segsum(True)
_sc_segsum = _make_sc_segsum(False)


def _tc1_body(x_ref, s_ref, c_ref, At_ref, Bt_ref, b_ref, o_ref):
    cnt = c_ref[0][:, 0:1] + c_ref[1][:, 0:1]
    sblk = (s_ref[0][...].astype(jnp.float32)
            + s_ref[1][...].astype(jnp.float32))
    inv = jnp.where(cnt > 0.0, 1.0 / jnp.maximum(cnt, 1.0), 0.0)
    mm = jnp.dot(x_ref[...], At_ref[...], preferred_element_type=jnp.float32)
    mm2 = jnp.dot(sblk * inv, Bt_ref[...], preferred_element_type=jnp.float32)
    o_ref[...] = jnp.where(cnt > 0.0, mm + b_ref[...] + mm2,
                           0.0).astype(jnp.bfloat16)


def _tc2_body(h_ref, s_ref, c_ref, x_ref, At_ref, Bt_ref, b_ref, o_ref):
    cnt = c_ref[0][:, 0:1] + c_ref[1][:, 0:1]
    sblk = (s_ref[0][...].astype(jnp.float32)
            + s_ref[1][...].astype(jnp.float32))
    inv = jnp.where(cnt > 0.0, 1.0 / jnp.maximum(cnt, 1.0), 0.0)
    h32 = h_ref[...].astype(jnp.float32)
    mm = jnp.dot(h32, At_ref[...], preferred_element_type=jnp.float32)
    mm2 = jnp.dot(sblk * inv, Bt_ref[...], preferred_element_type=jnp.float32)
    o_ref[...] = jnp.where(cnt > 0.0, mm + b_ref[...] + mm2, 0.0) + x_ref[...]


_W_SPEC = pl.BlockSpec((D, D), lambda i: (0, 0))
_B_SPEC = pl.BlockSpec((1, D), lambda i: (0, 0))
_ROW_SPEC = pl.BlockSpec((BM, D), lambda i: (i, 0))
_S_SPEC = pl.BlockSpec((NC, BM, D), lambda i: (0, i, 0))
_C_SPEC = pl.BlockSpec((NC, BM, CW), lambda i: (0, i, 0))


def _tc1(x, s1, cnt_p, At, Bt, b2d):
    return pl.pallas_call(
        _tc1_body,
        grid=(N_NODES // BM,),
        in_specs=[_ROW_SPEC, _S_SPEC, _C_SPEC, _W_SPEC, _W_SPEC, _B_SPEC],
        out_specs=_ROW_SPEC,
        out_shape=jax.ShapeDtypeStruct((N_NODES, D), jnp.bfloat16),
    )(x, s1, cnt_p, At, Bt, b2d)


def _tc2(h, s2, cnt_p, x, At, Bt, b2d):
    return pl.pallas_call(
        _tc2_body,
        grid=(N_NODES // BM,),
        in_specs=[_ROW_SPEC, _S_SPEC, _C_SPEC, _ROW_SPEC,
                  _W_SPEC, _W_SPEC, _B_SPEC],
        out_specs=_ROW_SPEC,
        out_shape=jax.ShapeDtypeStruct((N_NODES, D), jnp.float32),
    )(h, s2, cnt_p, x, At, Bt, b2d)


def kernel(nodes_feat, edges_index, batch, W_in, b_in, W_out, b_out):
    x = nodes_feat
    src2d = edges_index[0].reshape(N_EDGES // CHUNK, CHUNK)
    dst2d = edges_index[1].reshape(N_EDGES // CHUNK, CHUNK)
    zfeat = jnp.zeros((N_PAD, D), jnp.bfloat16)
    zcnt = jnp.zeros((N_PAD, CW), jnp.float32)
    ones2d = jnp.ones((CHUNK, CW), jnp.float32)
    A1t = (W_in[:, :D] - W_in[:, D:]).T
    B1t = W_in[:, D:].T
    A2t = (W_out[:, :D] - W_out[:, D:]).T
    B2t = W_out[:, D:].T

    s1, cnt_p = _sc_segsum_cnt(x.astype(jnp.bfloat16), src2d, dst2d,
                               zfeat, zcnt, ones2d)
    h = _tc1(x, s1, cnt_p, A1t, B1t, b_in.reshape(1, D))
    (s2,) = _sc_segsum(h, src2d, dst2d, zfeat)
    return _tc2(h, s2, cnt_p, x, A2t, B2t, b_out.reshape(1, D))


# speedup vs baseline: 1.3261x; 1.1555x over previous
import functools

import jax
import jax.numpy as jnp
from jax import lax
from jax.experimental import pallas as pl
from jax.experimental.pallas import tpu as pltpu
from jax.experimental.pallas import tpu_sc as plsc

N_NODES = 10000
N_PAD = 10240
N_EDGES = 320000
D = 128
CW = 16

NC = 2
NS = 16
NW = NC * NS
EDGES_PER_TILE = N_EDGES // NW
CHUNK = 80
CHUNKS_PER_TILE = EDGES_PER_TILE // CHUNK
ROWS_PER_TILE = N_PAD // NS
NBUF = 5
ZR = 64
BM = 1000

_MESH = plsc.VectorSubcoreMesh(core_axis_name="c", subcore_axis_name="s")
_SC_PARAMS = pltpu.CompilerParams(use_tc_tiling_on_sc=False)


def _tile_ids():
    c = lax.axis_index("c")
    s = lax.axis_index("s")
    return c, s, s * NC + c


def _stage_idx(idx_hbm, idx_v, wid):
    pltpu.sync_copy(idx_hbm.at[pl.ds(wid * EDGES_PER_TILE, EDGES_PER_TILE)],
                    idx_v)


def _make_sc_segsum(with_cnt):
    out_types = [jax.ShapeDtypeStruct((NC, N_PAD, D), jnp.bfloat16)]
    scratch = [
        pltpu.VMEM((EDGES_PER_TILE,), jnp.int32),
        pltpu.VMEM((EDGES_PER_TILE,), jnp.int32),
        pltpu.VMEM((ZR, D), jnp.bfloat16),
        [pltpu.VMEM((CHUNK, D), jnp.bfloat16)] * NBUF,
        pltpu.VMEM_SHARED((N_PAD, D), jnp.bfloat16),
        [pltpu.SemaphoreType.DMA] * NBUF,
        [pltpu.SemaphoreType.DMA] * NBUF,
    ]
    if with_cnt:
        out_types.append(jax.ShapeDtypeStruct((NC, N_PAD, CW), jnp.float32))
        scratch += [
            pltpu.VMEM((CHUNK, CW), jnp.float32),
            pltpu.VMEM((ZR, CW), jnp.float32),
            pltpu.VMEM_SHARED((N_PAD, CW), jnp.float32),
        ]

    @functools.partial(
        pl.kernel,
        mesh=_MESH,
        compiler_params=_SC_PARAMS,
        out_type=out_types,
        scratch_types=scratch,
    )
    def sc_segsum(table_hbm, src_hbm, dst_hbm, *refs):
        if with_cnt:
            (out_hbm, cnt_hbm,
             src_v, dst_v, zrow_v, rows, acc, gs, ss,
             ones_v, zcnt_v, cacc) = refs
        else:
            out_hbm, src_v, dst_v, zrow_v, rows, acc, gs, ss = refs
        c, s, wid = _tile_ids()
        row0 = s * ROWS_PER_TILE
        _stage_idx(src_hbm, src_v, wid)
        _stage_idx(dst_hbm, dst_v, wid)

        def _fill(i, carry):
            for k in range(D // 32):
                zrow_v[i, pl.ds(k * 32, 32)] = jnp.zeros((32,), jnp.bfloat16)
            if with_cnt:
                @pl.when(i < CHUNK)
                def _():
                    ones_v[i, pl.ds(0, CW)] = jnp.ones((CW,), jnp.float32)
                @pl.when(i < ZR)
                def _():
                    zcnt_v[i, pl.ds(0, CW)] = jnp.zeros((CW,), jnp.float32)
            return carry
        lax.fori_loop(0, max(ZR, CHUNK), _fill, 0)

        def _zero(k, carry):
            pltpu.sync_copy(zrow_v, acc.at[pl.ds(row0 + k * ZR, ZR)])
            if with_cnt:
                pltpu.sync_copy(zcnt_v, cacc.at[pl.ds(row0 + k * ZR, ZR)])
            return carry
        lax.fori_loop(0, ROWS_PER_TILE // ZR, _zero, 0)
        plsc.subcore_barrier()

        def _idx(j):
            return lambda ref: ref.at[pl.ds(j * CHUNK, CHUNK)]

        for b in range(NBUF - 1):
            pltpu.async_copy(table_hbm.at[_idx(b)(src_v)], rows[b], gs[b])

        def _group(i, carry):
            for b in range(NBUF):
                j = i * NBUF + b
                jn = j + NBUF - 1
                bn = (NBUF - 1 + b) % NBUF

                @pl.when(j > 0)
                def _():
                    pltpu.make_async_copy(rows[bn],
                                          acc.at[_idx(j - 1)(dst_v)],
                                          ss[bn]).wait()

                @pl.when(jn < CHUNKS_PER_TILE)
                def _():
                    pltpu.async_copy(table_hbm.at[_idx(jn)(src_v)], rows[bn],
                                     gs[bn])
                pltpu.make_async_copy(table_hbm.at[_idx(j)(src_v)], rows[b],
                                      gs[b]).wait()
                pltpu.async_copy(rows[b], acc.at[_idx(j)(dst_v)], ss[b],
                                 add=True)
                if with_cnt:
                    pltpu.sync_copy(ones_v, cacc.at[_idx(j)(dst_v)], add=True)
            return carry
        lax.fori_loop(0, CHUNKS_PER_TILE // NBUF, _group, 0)
        lastb = (CHUNKS_PER_TILE - 1) % NBUF
        pltpu.make_async_copy(rows[lastb],
                              acc.at[_idx(CHUNKS_PER_TILE - 1)(dst_v)],
                              ss[lastb]).wait()
        plsc.subcore_barrier()

        pltpu.sync_copy(acc.at[pl.ds(row0, ROWS_PER_TILE)],
                        out_hbm.at[c, pl.ds(row0, ROWS_PER_TILE)])
        if with_cnt:
            pltpu.sync_copy(cacc.at[pl.ds(row0, ROWS_PER_TILE)],
                            cnt_hbm.at[c, pl.ds(row0, ROWS_PER_TILE)])

    return sc_segsum


_sc_segsum_cnt = _make_sc_segsum(True)
_sc_segsum = _make_sc_segsum(False)


def _tc1_body(x_ref, s_ref, c_ref, At_ref, Bt_ref, b_ref, o_ref):
    cnt = c_ref[0][:, 0:1] + c_ref[1][:, 0:1]
    sblk = (s_ref[0][...].astype(jnp.float32)
            + s_ref[1][...].astype(jnp.float32))
    inv = jnp.where(cnt > 0.0, 1.0 / jnp.maximum(cnt, 1.0), 0.0)
    mm = jnp.dot(x_ref[...], At_ref[...], preferred_element_type=jnp.float32)
    mm2 = jnp.dot(sblk * inv, Bt_ref[...], preferred_element_type=jnp.float32)
    o_ref[...] = jnp.where(cnt > 0.0, mm + b_ref[...] + mm2,
                           0.0).astype(jnp.bfloat16)


def _tc2_body(h_ref, s_ref, c_ref, x_ref, At_ref, Bt_ref, b_ref, o_ref):
    cnt = c_ref[0][:, 0:1] + c_ref[1][:, 0:1]
    sblk = (s_ref[0][...].astype(jnp.float32)
            + s_ref[1][...].astype(jnp.float32))
    inv = jnp.where(cnt > 0.0, 1.0 / jnp.maximum(cnt, 1.0), 0.0)
    h32 = h_ref[...].astype(jnp.float32)
    mm = jnp.dot(h32, At_ref[...], preferred_element_type=jnp.float32)
    mm2 = jnp.dot(sblk * inv, Bt_ref[...], preferred_element_type=jnp.float32)
    o_ref[...] = jnp.where(cnt > 0.0, mm + b_ref[...] + mm2, 0.0) + x_ref[...]


_W_SPEC = pl.BlockSpec((D, D), lambda i: (0, 0))
_B_SPEC = pl.BlockSpec((1, D), lambda i: (0, 0))
_ROW_SPEC = pl.BlockSpec((BM, D), lambda i: (i, 0))
_S_SPEC = pl.BlockSpec((NC, BM, D), lambda i: (0, i, 0))
_C_SPEC = pl.BlockSpec((NC, BM, CW), lambda i: (0, i, 0))


def _tc1(x, s1, cnt_p, At, Bt, b2d):
    return pl.pallas_call(
        _tc1_body,
        grid=(N_NODES // BM,),
        in_specs=[_ROW_SPEC, _S_SPEC, _C_SPEC, _W_SPEC, _W_SPEC, _B_SPEC],
        out_specs=_ROW_SPEC,
        out_shape=jax.ShapeDtypeStruct((N_NODES, D), jnp.bfloat16),
    )(x, s1, cnt_p, At, Bt, b2d)


def _tc2(h, s2, cnt_p, x, At, Bt, b2d):
    return pl.pallas_call(
        _tc2_body,
        grid=(N_NODES // BM,),
        in_specs=[_ROW_SPEC, _S_SPEC, _C_SPEC, _ROW_SPEC,
                  _W_SPEC, _W_SPEC, _B_SPEC],
        out_specs=_ROW_SPEC,
        out_shape=jax.ShapeDtypeStruct((N_NODES, D), jnp.float32),
    )(h, s2, cnt_p, x, At, Bt, b2d)


def kernel(nodes_feat, edges_index, batch, W_in, b_in, W_out, b_out):
    x = nodes_feat
    src1d = edges_index[0]
    dst1d = edges_index[1]
    A1t = (W_in[:, :D] - W_in[:, D:]).T
    B1t = W_in[:, D:].T
    A2t = (W_out[:, :D] - W_out[:, D:]).T
    B2t = W_out[:, D:].T

    s1, cnt_p = _sc_segsum_cnt(x.astype(jnp.bfloat16), src1d, dst1d)
    h = _tc1(x, s1, cnt_p, A1t, B1t, b_in.reshape(1, D))
    (s2,) = _sc_segsum(h, src1d, dst1d)
    return _tc2(h, s2, cnt_p, x, A2t, B2t, b_out.reshape(1, D))


# speedup vs baseline: 1.3964x; 1.0530x over previous
import functools

import jax
import jax.numpy as jnp
from jax import lax
from jax.experimental import pallas as pl
from jax.experimental.pallas import tpu as pltpu
from jax.experimental.pallas import tpu_sc as plsc

N_NODES = 10000
N_PAD = 10240
N_EDGES = 320000
D = 128
CW = 16

NC = 2
NS = 16
NW = NC * NS
EDGES_PER_TILE = N_EDGES // NW
CHUNK = 80
CHUNKS_PER_TILE = EDGES_PER_TILE // CHUNK
ROWS_PER_TILE = N_PAD // NS
NBUF = 5
ZR = 64
BM = 2000

_MESH = plsc.VectorSubcoreMesh(core_axis_name="c", subcore_axis_name="s")
_SC_PARAMS = pltpu.CompilerParams(use_tc_tiling_on_sc=False)


def _tile_ids():
    c = lax.axis_index("c")
    s = lax.axis_index("s")
    return c, s, s * NC + c


def _stage_idx(edges_hbm, row, idx_v, wid):
    pltpu.sync_copy(
        edges_hbm.at[row, pl.ds(wid * EDGES_PER_TILE, EDGES_PER_TILE)], idx_v)


def _make_sc_segsum(with_cnt):
    out_types = [jax.ShapeDtypeStruct((NC, N_PAD, D), jnp.bfloat16)]
    scratch = [
        pltpu.VMEM((EDGES_PER_TILE,), jnp.int32),
        pltpu.VMEM((EDGES_PER_TILE,), jnp.int32),
        pltpu.VMEM((ZR, D), jnp.bfloat16),
        [pltpu.VMEM((CHUNK, D), jnp.bfloat16)] * NBUF,
        pltpu.VMEM_SHARED((N_PAD, D), jnp.bfloat16),
        [pltpu.SemaphoreType.DMA] * NBUF,
        [pltpu.SemaphoreType.DMA] * NBUF,
        pltpu.SemaphoreType.DMA,
    ]
    if with_cnt:
        out_types.append(jax.ShapeDtypeStruct((NC, N_PAD, CW), jnp.float32))
        scratch += [
            pltpu.VMEM((CHUNK, CW), jnp.float32),
            pltpu.VMEM((ZR, CW), jnp.float32),
            pltpu.VMEM_SHARED((N_PAD, CW), jnp.float32),
        ]

    @functools.partial(
        pl.kernel,
        mesh=_MESH,
        compiler_params=_SC_PARAMS,
        out_type=out_types,
        scratch_types=scratch,
    )
    def sc_segsum(table_hbm, edges_hbm, *refs):
        if with_cnt:
            (out_hbm, cnt_hbm,
             src_v, dst_v, zrow_v, rows, acc, gs, ss, cs,
             ones_v, zcnt_v, cacc) = refs
        else:
            out_hbm, src_v, dst_v, zrow_v, rows, acc, gs, ss, cs = refs
        c, s, wid = _tile_ids()
        row0 = s * ROWS_PER_TILE
        _stage_idx(edges_hbm, 0, src_v, wid)
        _stage_idx(edges_hbm, 1, dst_v, wid)

        def _fill(i, carry):
            for k in range(D // 32):
                zrow_v[i, pl.ds(k * 32, 32)] = jnp.zeros((32,), jnp.bfloat16)
            if with_cnt:
                @pl.when(i < CHUNK)
                def _():
                    ones_v[i, pl.ds(0, CW)] = jnp.ones((CW,), jnp.float32)
                @pl.when(i < ZR)
                def _():
                    zcnt_v[i, pl.ds(0, CW)] = jnp.zeros((CW,), jnp.float32)
            return carry
        lax.fori_loop(0, max(ZR, CHUNK), _fill, 0)

        def _zero(k, carry):
            pltpu.sync_copy(zrow_v, acc.at[pl.ds(row0 + k * ZR, ZR)])
            if with_cnt:
                pltpu.sync_copy(zcnt_v, cacc.at[pl.ds(row0 + k * ZR, ZR)])
            return carry
        lax.fori_loop(0, ROWS_PER_TILE // ZR, _zero, 0)
        plsc.subcore_barrier()

        def _idx(j):
            return lambda ref: ref.at[pl.ds(j * CHUNK, CHUNK)]

        for b in range(NBUF - 1):
            pltpu.async_copy(table_hbm.at[_idx(b)(src_v)], rows[b], gs[b])

        def _group(i, carry):
            for b in range(NBUF):
                j = i * NBUF + b
                jn = j + NBUF - 1
                bn = (NBUF - 1 + b) % NBUF

                @pl.when(j > 0)
                def _():
                    pltpu.make_async_copy(rows[bn],
                                          acc.at[_idx(j - 1)(dst_v)],
                                          ss[bn]).wait()

                @pl.when(jn < CHUNKS_PER_TILE)
                def _():
                    pltpu.async_copy(table_hbm.at[_idx(jn)(src_v)], rows[bn],
                                     gs[bn])
                pltpu.make_async_copy(table_hbm.at[_idx(j)(src_v)], rows[b],
                                      gs[b]).wait()
                pltpu.async_copy(rows[b], acc.at[_idx(j)(dst_v)], ss[b],
                                 add=True)
                if with_cnt:
                    pltpu.async_copy(ones_v, cacc.at[_idx(j)(dst_v)], cs,
                                     add=True)

                    @pl.when(j > 0)
                    def _():
                        pltpu.make_async_copy(ones_v,
                                              cacc.at[_idx(j - 1)(dst_v)],
                                              cs).wait()
            return carry
        lax.fori_loop(0, CHUNKS_PER_TILE // NBUF, _group, 0)
        lastb = (CHUNKS_PER_TILE - 1) % NBUF
        pltpu.make_async_copy(rows[lastb],
                              acc.at[_idx(CHUNKS_PER_TILE - 1)(dst_v)],
                              ss[lastb]).wait()
        if with_cnt:
            pltpu.make_async_copy(ones_v,
                                  cacc.at[_idx(CHUNKS_PER_TILE - 1)(dst_v)],
                                  cs).wait()
        plsc.subcore_barrier()

        pltpu.sync_copy(acc.at[pl.ds(row0, ROWS_PER_TILE)],
                        out_hbm.at[c, pl.ds(row0, ROWS_PER_TILE)])
        if with_cnt:
            pltpu.sync_copy(cacc.at[pl.ds(row0, ROWS_PER_TILE)],
                            cnt_hbm.at[c, pl.ds(row0, ROWS_PER_TILE)])

    return sc_segsum


_sc_segsum_cnt = _make_sc_segsum(True)
_sc_segsum = _make_sc_segsum(False)


def _tc1_body(x_ref, s_ref, c_ref, At_ref, Bt_ref, b_ref, o_ref):
    cnt = c_ref[0][:, 0:1] + c_ref[1][:, 0:1]
    sblk = (s_ref[0][...].astype(jnp.float32)
            + s_ref[1][...].astype(jnp.float32))
    inv = jnp.where(cnt > 0.0, 1.0 / jnp.maximum(cnt, 1.0), 0.0)
    mm = jnp.dot(x_ref[...], At_ref[...], preferred_element_type=jnp.float32)
    mm2 = jnp.dot(sblk * inv, Bt_ref[...], preferred_element_type=jnp.float32)
    o_ref[...] = jnp.where(cnt > 0.0, mm + b_ref[...] + mm2,
                           0.0).astype(jnp.bfloat16)


def _tc2_body(h_ref, s_ref, c_ref, x_ref, At_ref, Bt_ref, b_ref, o_ref):
    cnt = c_ref[0][:, 0:1] + c_ref[1][:, 0:1]
    sblk = (s_ref[0][...].astype(jnp.float32)
            + s_ref[1][...].astype(jnp.float32))
    inv = jnp.where(cnt > 0.0, 1.0 / jnp.maximum(cnt, 1.0), 0.0)
    h32 = h_ref[...].astype(jnp.float32)
    mm = jnp.dot(h32, At_ref[...], preferred_element_type=jnp.float32)
    mm2 = jnp.dot(sblk * inv, Bt_ref[...], preferred_element_type=jnp.float32)
    o_ref[...] = jnp.where(cnt > 0.0, mm + b_ref[...] + mm2, 0.0) + x_ref[...]


_W_SPEC = pl.BlockSpec((D, D), lambda i: (0, 0))
_B_SPEC = pl.BlockSpec((1, D), lambda i: (0, 0))
_ROW_SPEC = pl.BlockSpec((BM, D), lambda i: (i, 0))
_S_SPEC = pl.BlockSpec((NC, BM, D), lambda i: (0, i, 0))
_C_SPEC = pl.BlockSpec((NC, BM, CW), lambda i: (0, i, 0))


def _tc1(x, s1, cnt_p, At, Bt, b2d):
    return pl.pallas_call(
        _tc1_body,
        grid=(N_NODES // BM,),
        in_specs=[_ROW_SPEC, _S_SPEC, _C_SPEC, _W_SPEC, _W_SPEC, _B_SPEC],
        out_specs=_ROW_SPEC,
        out_shape=jax.ShapeDtypeStruct((N_NODES, D), jnp.bfloat16),
    )(x, s1, cnt_p, At, Bt, b2d)


def _tc2(h, s2, cnt_p, x, At, Bt, b2d):
    return pl.pallas_call(
        _tc2_body,
        grid=(N_NODES // BM,),
        in_specs=[_ROW_SPEC, _S_SPEC, _C_SPEC, _ROW_SPEC,
                  _W_SPEC, _W_SPEC, _B_SPEC],
        out_specs=_ROW_SPEC,
        out_shape=jax.ShapeDtypeStruct((N_NODES, D), jnp.float32),
    )(h, s2, cnt_p, x, At, Bt, b2d)


def kernel(nodes_feat, edges_index, batch, W_in, b_in, W_out, b_out):
    x = nodes_feat
    A1t = (W_in[:, :D] - W_in[:, D:]).T
    B1t = W_in[:, D:].T
    A2t = (W_out[:, :D] - W_out[:, D:]).T
    B2t = W_out[:, D:].T

    s1, cnt_p = _sc_segsum_cnt(x.astype(jnp.bfloat16), edges_index)
    h = _tc1(x, s1, cnt_p, A1t, B1t, b_in.reshape(1, D))
    (s2,) = _sc_segsum(h, edges_index)
    return _tc2(h, s2, cnt_p, x, A2t, B2t, b_out.reshape(1, D))


# speedup vs baseline: 1.4993x; 1.0737x over previous
import functools

import jax
import jax.numpy as jnp
from jax import lax
from jax.experimental import pallas as pl
from jax.experimental.pallas import tpu as pltpu
from jax.experimental.pallas import tpu_sc as plsc

N_NODES = 10000
N_PAD = 10240
N_EDGES = 320000
D = 128
CW = 16

NC = 2
NS = 16
NW = NC * NS
EDGES_PER_TILE = N_EDGES // NW
CHUNK = 80
CHUNKS_PER_TILE = EDGES_PER_TILE // CHUNK
ROWS_PER_TILE = N_PAD // NS
NBUF = 5
ZR = 64
BM = 2000

_MESH = plsc.VectorSubcoreMesh(core_axis_name="c", subcore_axis_name="s")
_SC_PARAMS = pltpu.CompilerParams(use_tc_tiling_on_sc=False)


def _tile_ids():
    c = lax.axis_index("c")
    s = lax.axis_index("s")
    return c, s, s * NC + c


def _stage_idx(edges_hbm, row, idx_v, wid):
    pltpu.sync_copy(
        edges_hbm.at[row, pl.ds(wid * EDGES_PER_TILE, EDGES_PER_TILE)], idx_v)


def _make_sc_segsum(with_cnt):
    out_types = [jax.ShapeDtypeStruct((N_PAD, NC * D), jnp.bfloat16)]
    scratch = [
        pltpu.VMEM((EDGES_PER_TILE,), jnp.int32),
        pltpu.VMEM((EDGES_PER_TILE,), jnp.int32),
        pltpu.VMEM((ZR, D), jnp.bfloat16),
        [pltpu.VMEM((CHUNK, D), jnp.bfloat16)] * NBUF,
        pltpu.VMEM_SHARED((N_PAD, D), jnp.bfloat16),
        [pltpu.SemaphoreType.DMA] * NBUF,
        [pltpu.SemaphoreType.DMA] * NBUF,
        pltpu.SemaphoreType.DMA,
    ]
    if with_cnt:
        out_types.append(jax.ShapeDtypeStruct((N_PAD, NC * CW), jnp.float32))
        scratch += [
            pltpu.VMEM((CHUNK, CW), jnp.float32),
            pltpu.VMEM((ZR, CW), jnp.float32),
            pltpu.VMEM_SHARED((N_PAD, CW), jnp.float32),
        ]

    @functools.partial(
        pl.kernel,
        mesh=_MESH,
        compiler_params=_SC_PARAMS,
        out_type=out_types,
        scratch_types=scratch,
    )
    def sc_segsum(table_hbm, edges_hbm, *refs):
        if with_cnt:
            (out_hbm, cnt_hbm,
             src_v, dst_v, zrow_v, rows, acc, gs, ss, cs,
             ones_v, zcnt_v, cacc) = refs
        else:
            out_hbm, src_v, dst_v, zrow_v, rows, acc, gs, ss, cs = refs
        c, s, wid = _tile_ids()
        row0 = s * ROWS_PER_TILE
        _stage_idx(edges_hbm, 0, src_v, wid)
        _stage_idx(edges_hbm, 1, dst_v, wid)

        def _fill(i, carry):
            for k in range(D // 32):
                zrow_v[i, pl.ds(k * 32, 32)] = jnp.zeros((32,), jnp.bfloat16)
            if with_cnt:
                @pl.when(i < CHUNK)
                def _():
                    ones_v[i, pl.ds(0, CW)] = jnp.ones((CW,), jnp.float32)
                @pl.when(i < ZR)
                def _():
                    zcnt_v[i, pl.ds(0, CW)] = jnp.zeros((CW,), jnp.float32)
            return carry
        lax.fori_loop(0, max(ZR, CHUNK), _fill, 0)

        def _zero(k, carry):
            pltpu.sync_copy(zrow_v, acc.at[pl.ds(row0 + k * ZR, ZR)])
            if with_cnt:
                pltpu.sync_copy(zcnt_v, cacc.at[pl.ds(row0 + k * ZR, ZR)])
            return carry
        lax.fori_loop(0, ROWS_PER_TILE // ZR, _zero, 0)
        plsc.subcore_barrier()

        def _idx(j):
            return lambda ref: ref.at[pl.ds(j * CHUNK, CHUNK)]

        for b in range(NBUF - 1):
            pltpu.async_copy(table_hbm.at[_idx(b)(src_v)], rows[b], gs[b])

        def _group(i, carry):
            for b in range(NBUF):
                j = i * NBUF + b
                jn = j + NBUF - 1
                bn = (NBUF - 1 + b) % NBUF

                @pl.when(j > 0)
                def _():
                    pltpu.make_async_copy(rows[bn],
                                          acc.at[_idx(j - 1)(dst_v)],
                                          ss[bn]).wait()

                @pl.when(jn < CHUNKS_PER_TILE)
                def _():
                    pltpu.async_copy(table_hbm.at[_idx(jn)(src_v)], rows[bn],
                                     gs[bn])
                pltpu.make_async_copy(table_hbm.at[_idx(j)(src_v)], rows[b],
                                      gs[b]).wait()
                pltpu.async_copy(rows[b], acc.at[_idx(j)(dst_v)], ss[b],
                                 add=True)
                if with_cnt:
                    pltpu.async_copy(ones_v, cacc.at[_idx(j)(dst_v)], cs,
                                     add=True)

                    @pl.when(j > 0)
                    def _():
                        pltpu.make_async_copy(ones_v,
                                              cacc.at[_idx(j - 1)(dst_v)],
                                              cs).wait()
            return carry
        lax.fori_loop(0, CHUNKS_PER_TILE // NBUF, _group, 0)
        lastb = (CHUNKS_PER_TILE - 1) % NBUF
        pltpu.make_async_copy(rows[lastb],
                              acc.at[_idx(CHUNKS_PER_TILE - 1)(dst_v)],
                              ss[lastb]).wait()
        if with_cnt:
            pltpu.make_async_copy(ones_v,
                                  cacc.at[_idx(CHUNKS_PER_TILE - 1)(dst_v)],
                                  cs).wait()
        plsc.subcore_barrier()

        pltpu.sync_copy(acc.at[pl.ds(row0, ROWS_PER_TILE)],
                        out_hbm.at[pl.ds(row0, ROWS_PER_TILE),
                                   pl.ds(c * D, D)])
        if with_cnt:
            pltpu.sync_copy(cacc.at[pl.ds(row0, ROWS_PER_TILE)],
                            cnt_hbm.at[pl.ds(row0, ROWS_PER_TILE),
                                       pl.ds(c * CW, CW)])

    return sc_segsum


_sc_segsum_cnt = _make_sc_segsum(True)
_sc_segsum = _make_sc_segsum(False)


def _tc1_body(x_ref, s_ref, c_ref, At_ref, Bt_ref, b_ref, o_ref):
    cnt = c_ref[:, 0:1] + c_ref[:, CW:CW + 1]
    sblk = (s_ref[:, :D].astype(jnp.float32)
            + s_ref[:, D:].astype(jnp.float32))
    inv = jnp.where(cnt > 0.0, 1.0 / jnp.maximum(cnt, 1.0), 0.0)
    mm = jnp.dot(x_ref[...], At_ref[...], preferred_element_type=jnp.float32)
    mm2 = jnp.dot(sblk * inv, Bt_ref[...], preferred_element_type=jnp.float32)
    o_ref[...] = jnp.where(cnt > 0.0, mm + b_ref[...] + mm2,
                           0.0).astype(jnp.bfloat16)


def _tc2_body(h_ref, s_ref, c_ref, x_ref, At_ref, Bt_ref, b_ref, o_ref):
    cnt = c_ref[:, 0:1] + c_ref[:, CW:CW + 1]
    sblk = (s_ref[:, :D].astype(jnp.float32)
            + s_ref[:, D:].astype(jnp.float32))
    inv = jnp.where(cnt > 0.0, 1.0 / jnp.maximum(cnt, 1.0), 0.0)
    h32 = h_ref[...].astype(jnp.float32)
    mm = jnp.dot(h32, At_ref[...], preferred_element_type=jnp.float32)
    mm2 = jnp.dot(sblk * inv, Bt_ref[...], preferred_element_type=jnp.float32)
    o_ref[...] = jnp.where(cnt > 0.0, mm + b_ref[...] + mm2, 0.0) + x_ref[...]


_W_SPEC = pl.BlockSpec((D, D), lambda i: (0, 0))
_B_SPEC = pl.BlockSpec((1, D), lambda i: (0, 0))
_ROW_SPEC = pl.BlockSpec((BM, D), lambda i: (i, 0))
_S_SPEC = pl.BlockSpec((BM, NC * D), lambda i: (i, 0))
_C_SPEC = pl.BlockSpec((BM, NC * CW), lambda i: (i, 0))


def _tc1(x, s1, cnt_p, At, Bt, b2d):
    return pl.pallas_call(
        _tc1_body,
        grid=(N_NODES // BM,),
        in_specs=[_ROW_SPEC, _S_SPEC, _C_SPEC, _W_SPEC, _W_SPEC, _B_SPEC],
        out_specs=_ROW_SPEC,
        out_shape=jax.ShapeDtypeStruct((N_NODES, D), jnp.bfloat16),
    )(x, s1, cnt_p, At, Bt, b2d)


def _tc2(h, s2, cnt_p, x, At, Bt, b2d):
    return pl.pallas_call(
        _tc2_body,
        grid=(N_NODES // BM,),
        in_specs=[_ROW_SPEC, _S_SPEC, _C_SPEC, _ROW_SPEC,
                  _W_SPEC, _W_SPEC, _B_SPEC],
        out_specs=_ROW_SPEC,
        out_shape=jax.ShapeDtypeStruct((N_NODES, D), jnp.float32),
    )(h, s2, cnt_p, x, At, Bt, b2d)


def kernel(nodes_feat, edges_index, batch, W_in, b_in, W_out, b_out):
    x = nodes_feat
    A1t = (W_in[:, :D] - W_in[:, D:]).T
    B1t = W_in[:, D:].T
    A2t = (W_out[:, :D] - W_out[:, D:]).T
    B2t = W_out[:, D:].T

    s1, cnt_p = _sc_segsum_cnt(x.astype(jnp.bfloat16), edges_index)
    h = _tc1(x, s1, cnt_p, A1t, B1t, b_in.reshape(1, D))
    (s2,) = _sc_segsum(h, edges_index)
    return _tc2(h, s2, cnt_p, x, A2t, B2t, b_out.reshape(1, D))
